# Initial kernel scaffold; baseline (speedup 1.0000x reference)
import jax
import jax.numpy as jnp
from jax.experimental import pallas as pl


def kernel(x, w1, s1, b1, w2, s2, b2):
    raise NotImplementedError("write your pallas kernel here")


# speedup vs baseline: 1.0348x; 1.0348x over previous
import functools

import jax
import jax.numpy as jnp
from jax.experimental import pallas as pl
from jax.experimental.pallas import tpu as pltpu


def _fused_block_kernel(x_ref, w1_ref, b1_ref, w2_ref, b2_ref, o_ref, mid_ref,
                        *, D, H, W, C):
    M = D * H * W
    taps = [(kd, kh, kw) for kd in range(3) for kh in range(3) for kw in range(3)]

    def im2col(window):
        cols = [window(kd, kh, kw).reshape(M, C) for kd, kh, kw in taps]
        return jnp.concatenate(cols, axis=-1)

    lhs1 = im2col(lambda kd, kh, kw: x_ref[0, kd:kd + D, kh:kh + H, kw:kw + W, :])
    y = jnp.dot(lhs1, w1_ref[...], preferred_element_type=jnp.float32)
    y = jnp.maximum(y + b1_ref[...], 0.0).astype(jnp.bfloat16)
    mid_ref[...] = jnp.zeros_like(mid_ref)
    mid_ref[1:D + 1, 1:H + 1, 1:W + 1, :] = y.reshape(D, H, W, C)

    lhs2 = im2col(lambda kd, kh, kw: mid_ref[kd:kd + D, kh:kh + H, kw:kw + W, :])
    z = jnp.dot(lhs2, w2_ref[...], preferred_element_type=jnp.float32)
    res = x_ref[0, 1:D + 1, 1:H + 1, 1:W + 1, :].reshape(M, C).astype(jnp.float32)
    z = jnp.maximum(z + b2_ref[...] + res, 0.0)
    o_ref[...] = z.reshape(1, D, H, W, C)


def kernel(x, w1, s1, b1, w2, s2, b2):
    N, D, H, W, C = x.shape
    K = 27 * C

    xb = x.astype(jnp.bfloat16)
    x_pad = jnp.pad(xb, ((0, 0), (1, 1), (1, 1), (1, 1), (0, 0)))
    w1f = (w1 * s1).astype(jnp.bfloat16).reshape(K, C)
    w2f = (w2 * s2).astype(jnp.bfloat16).reshape(K, C)
    b1f = b1.reshape(1, C).astype(jnp.float32)
    b2f = b2.reshape(1, C).astype(jnp.float32)

    body = functools.partial(_fused_block_kernel, D=D, H=H, W=W, C=C)

    flops = 2 * 2 * N * D * H * W * 27 * C * C + 4 * N * D * H * W * C
    bytes_accessed = (x_pad.size * 2 + 2 * K * C * 2 + N * D * H * W * C * 4)

    return pl.pallas_call(
        body,
        out_shape=jax.ShapeDtypeStruct((N, D, H, W, C), x.dtype),
        grid=(N,),
        in_specs=[
            pl.BlockSpec((1, D + 2, H + 2, W + 2, C), lambda n: (n, 0, 0, 0, 0)),
            pl.BlockSpec((K, C), lambda n: (0, 0)),
            pl.BlockSpec((1, C), lambda n: (0, 0)),
            pl.BlockSpec((K, C), lambda n: (0, 0)),
            pl.BlockSpec((1, C), lambda n: (0, 0)),
        ],
        out_specs=pl.BlockSpec((1, D, H, W, C), lambda n: (n, 0, 0, 0, 0)),
        scratch_shapes=[pltpu.VMEM((D + 2, H + 2, W + 2, C), jnp.bfloat16)],
        compiler_params=pltpu.CompilerParams(
            dimension_semantics=("parallel",),
            vmem_limit_bytes=56 * 1024 * 1024),
        cost_estimate=pl.CostEstimate(
            flops=int(flops), transcendentals=0, bytes_accessed=int(bytes_accessed)),
    )(x_pad, w1f, b1f, w2f, b2f)


# speedup vs baseline: 1.2170x; 1.1761x over previous
import functools

import jax
import jax.numpy as jnp
from jax import lax
from jax.experimental import pallas as pl
from jax.experimental.pallas import tpu as pltpu


def _fused_block_kernel(x_ref, w1_ref, b1_ref, w2_ref, b2_ref, o_ref,
                        b_ref, mid_ref, *, D, H, W, C):
    M = H * W

    def build_b(window):
        b_ref[...] = jnp.concatenate(
            [window(kw) for kw in range(3)], axis=-1)

    def conv_rows(w_ref, epilogue):
        def row(d, carry):
            acc = jnp.zeros((M, C), dtype=jnp.float32)
            for kd in range(3):
                for kh in range(3):
                    lhs = b_ref[pl.ds(d + kd, 1), kh:kh + H, :, :]
                    acc = acc + jnp.dot(
                        lhs.reshape(M, 3 * C), w_ref[kd, kh],
                        preferred_element_type=jnp.float32)
            epilogue(d, acc)
            return carry
        lax.fori_loop(0, D, row, 0)

    build_b(lambda kw: x_ref[0, :, :, kw:kw + W, :])
    mid_ref[...] = jnp.zeros_like(mid_ref)

    def epi1(d, acc):
        y = jnp.maximum(acc + b1_ref[...], 0.0).astype(jnp.bfloat16)
        mid_ref[pl.ds(d + 1, 1), 1:H + 1, 1:W + 1, :] = y.reshape(1, H, W, C)

    conv_rows(w1_ref, epi1)

    build_b(lambda kw: mid_ref[:, :, kw:kw + W, :])

    def epi2(d, acc):
        res = x_ref[0, pl.ds(d + 1, 1), 1:H + 1, 1:W + 1, :]
        z = acc + b2_ref[...] + res.reshape(M, C).astype(jnp.float32)
        z = jnp.maximum(z, 0.0)
        o_ref[0, pl.ds(d, 1)] = z.reshape(1, H, W, C)

    conv_rows(w2_ref, epi2)


def kernel(x, w1, s1, b1, w2, s2, b2):
    N, D, H, W, C = x.shape

    xb = x.astype(jnp.bfloat16)
    x_pad = jnp.pad(xb, ((0, 0), (1, 1), (1, 1), (1, 1), (0, 0)))
    w1f = (w1 * s1).astype(jnp.bfloat16).reshape(3, 3, 3 * C, C)
    w2f = (w2 * s2).astype(jnp.bfloat16).reshape(3, 3, 3 * C, C)
    b1f = b1.reshape(1, C).astype(jnp.float32)
    b2f = b2.reshape(1, C).astype(jnp.float32)

    body = functools.partial(_fused_block_kernel, D=D, H=H, W=W, C=C)

    flops = 2 * 2 * N * D * H * W * 27 * C * C + 4 * N * D * H * W * C
    bytes_accessed = (x_pad.size * 2 + 2 * 27 * C * C * 2 + N * D * H * W * C * 4)

    return pl.pallas_call(
        body,
        out_shape=jax.ShapeDtypeStruct((N, D, H, W, C), x.dtype),
        grid=(N,),
        in_specs=[
            pl.BlockSpec((1, D + 2, H + 2, W + 2, C), lambda n: (n, 0, 0, 0, 0)),
            pl.BlockSpec((3, 3, 3 * C, C), lambda n: (0, 0, 0, 0)),
            pl.BlockSpec((1, C), lambda n: (0, 0)),
            pl.BlockSpec((3, 3, 3 * C, C), lambda n: (0, 0, 0, 0)),
            pl.BlockSpec((1, C), lambda n: (0, 0)),
        ],
        out_specs=pl.BlockSpec((1, D, H, W, C), lambda n: (n, 0, 0, 0, 0)),
        scratch_shapes=[
            pltpu.VMEM((D + 2, H + 2, W, 3 * C), jnp.bfloat16),
            pltpu.VMEM((D + 2, H + 2, W + 2, C), jnp.bfloat16),
        ],
        compiler_params=pltpu.CompilerParams(
            dimension_semantics=("parallel",),
            vmem_limit_bytes=56 * 1024 * 1024),
        cost_estimate=pl.CostEstimate(
            flops=int(flops), transcendentals=0, bytes_accessed=int(bytes_accessed)),
    )(x_pad, w1f, b1f, w2f, b2f)


# speedup vs baseline: 1.8791x; 1.5440x over previous
import functools

import jax
import jax.numpy as jnp
from jax import lax
from jax.experimental import pallas as pl
from jax.experimental.pallas import tpu as pltpu


def _fused_block_kernel(x_ref, w1_ref, b1_ref, w2_ref, b2_ref, o_ref,
                        b_ref, mid_ref, *, D, H, W, C):
    M = H * W

    def build_b(window):
        b_ref[...] = jnp.concatenate(
            [window(kw) for kw in range(3)], axis=-1)

    def conv_rows(w_ref, epilogue):
        for d in range(D):
            acc = jnp.zeros((M, C), dtype=jnp.float32)
            for kd in range(3):
                for kh in range(3):
                    lhs = b_ref[d + kd, kh:kh + H, :, :]
                    acc = acc + jnp.dot(
                        lhs.reshape(M, 3 * C), w_ref[kd, kh],
                        preferred_element_type=jnp.float32)
            epilogue(d, acc)

    build_b(lambda kw: x_ref[0, :, :, kw:kw + W, :])
    mid_ref[...] = jnp.zeros_like(mid_ref)

    def epi1(d, acc):
        y = jnp.maximum(acc + b1_ref[...], 0.0).astype(jnp.bfloat16)
        mid_ref[d + 1, 1:H + 1, 1:W + 1, :] = y.reshape(H, W, C)

    conv_rows(w1_ref, epi1)

    build_b(lambda kw: mid_ref[:, :, kw:kw + W, :])

    def epi2(d, acc):
        res = x_ref[0, d + 1, 1:H + 1, 1:W + 1, :]
        z = acc + b2_ref[...] + res.reshape(M, C).astype(jnp.float32)
        z = jnp.maximum(z, 0.0)
        o_ref[0, d] = z.reshape(H, W, C)

    conv_rows(w2_ref, epi2)


def kernel(x, w1, s1, b1, w2, s2, b2):
    N, D, H, W, C = x.shape

    xb = x.astype(jnp.bfloat16)
    x_pad = jnp.pad(xb, ((0, 0), (1, 1), (1, 1), (1, 1), (0, 0)))
    w1f = (w1 * s1).astype(jnp.bfloat16).reshape(3, 3, 3 * C, C)
    w2f = (w2 * s2).astype(jnp.bfloat16).reshape(3, 3, 3 * C, C)
    b1f = b1.reshape(1, C).astype(jnp.float32)
    b2f = b2.reshape(1, C).astype(jnp.float32)

    body = functools.partial(_fused_block_kernel, D=D, H=H, W=W, C=C)

    flops = 2 * 2 * N * D * H * W * 27 * C * C + 4 * N * D * H * W * C
    bytes_accessed = (x_pad.size * 2 + 2 * 27 * C * C * 2 + N * D * H * W * C * 4)

    return pl.pallas_call(
        body,
        out_shape=jax.ShapeDtypeStruct((N, D, H, W, C), x.dtype),
        grid=(N,),
        in_specs=[
            pl.BlockSpec((1, D + 2, H + 2, W + 2, C), lambda n: (n, 0, 0, 0, 0)),
            pl.BlockSpec((3, 3, 3 * C, C), lambda n: (0, 0, 0, 0)),
            pl.BlockSpec((1, C), lambda n: (0, 0)),
            pl.BlockSpec((3, 3, 3 * C, C), lambda n: (0, 0, 0, 0)),
            pl.BlockSpec((1, C), lambda n: (0, 0)),
        ],
        out_specs=pl.BlockSpec((1, D, H, W, C), lambda n: (n, 0, 0, 0, 0)),
        scratch_shapes=[
            pltpu.VMEM((D + 2, H + 2, W, 3 * C), jnp.bfloat16),
            pltpu.VMEM((D + 2, H + 2, W + 2, C), jnp.bfloat16),
        ],
        compiler_params=pltpu.CompilerParams(
            dimension_semantics=("parallel",),
            vmem_limit_bytes=56 * 1024 * 1024),
        cost_estimate=pl.CostEstimate(
            flops=int(flops), transcendentals=0, bytes_accessed=int(bytes_accessed)),
    )(x_pad, w1f, b1f, w2f, b2f)


# speedup vs baseline: 2.2556x; 1.2004x over previous
import functools

import jax
import jax.numpy as jnp
from jax.experimental import pallas as pl
from jax.experimental.pallas import tpu as pltpu


def _fused_block_kernel(x_ref, w1_ref, b1_ref, w2_ref, b2_ref, o_ref,
                        b_ref, mid_ref, *, D, H, W, C):
    M = H * W

    def kw_stack(v):
        zrow = jnp.zeros(v.shape[:-2] + (1, C), dtype=v.dtype)
        s0 = jnp.concatenate([zrow, v[..., :W - 1, :]], axis=-2)
        s2 = jnp.concatenate([v[..., 1:, :], zrow], axis=-2)
        return jnp.concatenate([s0, v, s2], axis=-1)

    def conv_rows(w_ref, epilogue):
        for d in range(D):
            acc = jnp.zeros((M, C), dtype=jnp.float32)
            for kd in range(3):
                for kh in range(3):
                    lhs = b_ref[d + kd, kh:kh + H, :, :]
                    acc = acc + jnp.dot(
                        lhs.reshape(M, 3 * C), w_ref[kd, kh],
                        preferred_element_type=jnp.float32)
            epilogue(d, acc)

    zplane_d = jnp.zeros((1, H + 2, W, 3 * C), dtype=jnp.bfloat16)
    zplane_h = jnp.zeros((D + 2, 1, W, 3 * C), dtype=jnp.bfloat16)
    b_ref[0] = zplane_d[0]
    b_ref[D + 1] = zplane_d[0]
    b_ref[:, 0] = zplane_h[:, 0]
    b_ref[:, H + 1] = zplane_h[:, 0]
    b_ref[1:D + 1, 1:H + 1, :, :] = kw_stack(x_ref[0].astype(jnp.bfloat16))

    mid_ref[...] = jnp.zeros_like(mid_ref)

    def epi1(d, acc):
        y = jnp.maximum(acc + b1_ref[...], 0.0).astype(jnp.bfloat16)
        mid_ref[d + 1, 1:H + 1, :, :] = y.reshape(H, W, C)

    conv_rows(w1_ref, epi1)

    b_ref[...] = kw_stack(mid_ref[...])

    def epi2(d, acc):
        res = x_ref[0, d].reshape(M, C)
        z = jnp.maximum(acc + b2_ref[...] + res, 0.0)
        o_ref[0, d] = z.reshape(H, W, C)

    conv_rows(w2_ref, epi2)


def kernel(x, w1, s1, b1, w2, s2, b2):
    N, D, H, W, C = x.shape

    w1f = (w1 * s1).astype(jnp.bfloat16).reshape(3, 3, 3 * C, C)
    w2f = (w2 * s2).astype(jnp.bfloat16).reshape(3, 3, 3 * C, C)
    b1f = b1.reshape(1, C).astype(jnp.float32)
    b2f = b2.reshape(1, C).astype(jnp.float32)

    body = functools.partial(_fused_block_kernel, D=D, H=H, W=W, C=C)

    flops = 2 * 2 * N * D * H * W * 27 * C * C + 4 * N * D * H * W * C
    bytes_accessed = (x.size * 4 + 2 * 27 * C * C * 2 + N * D * H * W * C * 4)

    return pl.pallas_call(
        body,
        out_shape=jax.ShapeDtypeStruct((N, D, H, W, C), x.dtype),
        grid=(N,),
        in_specs=[
            pl.BlockSpec((1, D, H, W, C), lambda n: (n, 0, 0, 0, 0)),
            pl.BlockSpec((3, 3, 3 * C, C), lambda n: (0, 0, 0, 0)),
            pl.BlockSpec((1, C), lambda n: (0, 0)),
            pl.BlockSpec((3, 3, 3 * C, C), lambda n: (0, 0, 0, 0)),
            pl.BlockSpec((1, C), lambda n: (0, 0)),
        ],
        out_specs=pl.BlockSpec((1, D, H, W, C), lambda n: (n, 0, 0, 0, 0)),
        scratch_shapes=[
            pltpu.VMEM((D + 2, H + 2, W, 3 * C), jnp.bfloat16),
            pltpu.VMEM((D + 2, H + 2, W, C), jnp.bfloat16),
        ],
        compiler_params=pltpu.CompilerParams(
            dimension_semantics=("parallel",),
            vmem_limit_bytes=56 * 1024 * 1024),
        cost_estimate=pl.CostEstimate(
            flops=int(flops), transcendentals=0, bytes_accessed=int(bytes_accessed)),
    )(x, w1f, b1f, w2f, b2f)


# speedup vs baseline: 3.8780x; 1.7192x over previous
import functools

import jax
import jax.numpy as jnp
from jax.experimental import pallas as pl
from jax.experimental.pallas import tpu as pltpu


def _fused_block_kernel(x_ref, w1_ref, b1_ref, w2_ref, b2_ref, o_ref,
                        b_ref, mid_ref, *, D, H, W, C):
    M = H * W

    def kw_stack(v):
        zrow = jnp.zeros(v.shape[:-2] + (1, C), dtype=v.dtype)
        s0 = jnp.concatenate([zrow, v[..., :W - 1, :]], axis=-2)
        s2 = jnp.concatenate([v[..., 1:, :], zrow], axis=-2)
        return jnp.concatenate([s0, v, s2], axis=-1)

    def conv_rows(w_ref, epilogue):
        accs = {}

        def add(d, contrib):
            accs[d] = accs[d] + contrib if d in accs else contrib

        for dz in range(1, D + 1):
            lhs = b_ref[dz].reshape((H + 2) * W, 3 * C)
            r = jnp.dot(lhs, w_ref[...], preferred_element_type=jnp.float32)
            for kd in range(3):
                d = dz - kd
                if 0 <= d < D:
                    for kh in range(3):
                        j = kd * 3 + kh
                        add(d, r[kh * W:kh * W + M, j * C:(j + 1) * C])
            if dz - 2 >= 0:
                epilogue(dz - 2, accs.pop(dz - 2))
        epilogue(D - 1, accs.pop(D - 1))

    zplane_d = jnp.zeros((1, H + 2, W, 3 * C), dtype=jnp.bfloat16)
    zplane_h = jnp.zeros((D + 2, 1, W, 3 * C), dtype=jnp.bfloat16)
    b_ref[0] = zplane_d[0]
    b_ref[D + 1] = zplane_d[0]
    b_ref[:, 0] = zplane_h[:, 0]
    b_ref[:, H + 1] = zplane_h[:, 0]
    b_ref[1:D + 1, 1:H + 1, :, :] = kw_stack(x_ref[0].astype(jnp.bfloat16))

    mid_ref[...] = jnp.zeros_like(mid_ref)

    def epi1(d, acc):
        y = jnp.maximum(acc + b1_ref[...], 0.0).astype(jnp.bfloat16)
        mid_ref[d + 1, 1:H + 1, :, :] = y.reshape(H, W, C)

    conv_rows(w1_ref, epi1)

    b_ref[...] = kw_stack(mid_ref[...])

    def epi2(d, acc):
        res = x_ref[0, d].reshape(M, C)
        z = jnp.maximum(acc + b2_ref[...] + res, 0.0)
        o_ref[0, d] = z.reshape(H, W, C)

    conv_rows(w2_ref, epi2)


def kernel(x, w1, s1, b1, w2, s2, b2):
    N, D, H, W, C = x.shape

    w1f = (w1 * s1).astype(jnp.bfloat16).transpose(2, 3, 0, 1, 4).reshape(3 * C, 9 * C)
    w2f = (w2 * s2).astype(jnp.bfloat16).transpose(2, 3, 0, 1, 4).reshape(3 * C, 9 * C)
    b1f = b1.reshape(1, C).astype(jnp.float32)
    b2f = b2.reshape(1, C).astype(jnp.float32)

    body = functools.partial(_fused_block_kernel, D=D, H=H, W=W, C=C)

    flops = 2 * 2 * N * D * H * W * 27 * C * C + 4 * N * D * H * W * C
    bytes_accessed = (x.size * 4 + 2 * 27 * C * C * 2 + N * D * H * W * C * 4)

    return pl.pallas_call(
        body,
        out_shape=jax.ShapeDtypeStruct((N, D, H, W, C), x.dtype),
        grid=(N,),
        in_specs=[
            pl.BlockSpec((1, D, H, W, C), lambda n: (n, 0, 0, 0, 0)),
            pl.BlockSpec((3 * C, 9 * C), lambda n: (0, 0)),
            pl.BlockSpec((1, C), lambda n: (0, 0)),
            pl.BlockSpec((3 * C, 9 * C), lambda n: (0, 0)),
            pl.BlockSpec((1, C), lambda n: (0, 0)),
        ],
        out_specs=pl.BlockSpec((1, D, H, W, C), lambda n: (n, 0, 0, 0, 0)),
        scratch_shapes=[
            pltpu.VMEM((D + 2, H + 2, W, 3 * C), jnp.bfloat16),
            pltpu.VMEM((D + 2, H + 2, W, C), jnp.bfloat16),
        ],
        compiler_params=pltpu.CompilerParams(
            dimension_semantics=("parallel",),
            vmem_limit_bytes=56 * 1024 * 1024),
        cost_estimate=pl.CostEstimate(
            flops=int(flops), transcendentals=0, bytes_accessed=int(bytes_accessed)),
    )(x, w1f, b1f, w2f, b2f)
